# baseline (speedup 1.0000x reference)
import functools

import jax
import jax.numpy as jnp
from jax import lax
from jax.experimental import pallas as pl
from jax.experimental.pallas import tpu as pltpu
from jax.experimental.pallas import tpu_sc as plsc

_E = 64
_D = 768
_H = 1024
_S = 2048

_ALIGN = 8
_BM = 32
_HSPLIT = 2
_NW = 32
_P = 2816


def _router_body(x_ref, wg_ref, dest_ref, off_ref, cnt_ref):
    scores = lax.dot_general(x_ref[...], wg_ref[...],
                             (((1,), (1,)), ((), ())),
                             preferred_element_type=jnp.float32)
    idx = jnp.argmax(scores, axis=1).astype(jnp.int32)
    eids = lax.broadcasted_iota(jnp.int32, (1, _E), 1)
    onehot = (idx[:, None] == eids).astype(jnp.float32)
    r = lax.broadcasted_iota(jnp.int32, (_S, _S), 0)
    c = lax.broadcasted_iota(jnp.int32, (_S, _S), 1)
    tril = (c <= r).astype(jnp.float32)
    csum = lax.dot_general(tril, onehot, (((1,), (0,)), ((), ())),
                           preferred_element_type=jnp.float32)
    within = jnp.sum(csum * onehot, axis=1) - 1.0
    counts = csum[_S - 1, :].astype(jnp.int32)
    padded = (counts + (_ALIGN - 1)) // _ALIGN * _ALIGN
    re = lax.broadcasted_iota(jnp.int32, (_E, _E), 0)
    ce = lax.broadcasted_iota(jnp.int32, (_E, _E), 1)
    sut = (re < ce).astype(jnp.float32)
    poff = lax.dot_general(padded.astype(jnp.float32)[None, :], sut,
                           (((1,), (0,)), ((), ())),
                           preferred_element_type=jnp.float32)
    off_tok = jnp.sum(onehot * poff, axis=1)
    dest_ref[...] = (off_tok + within).astype(jnp.int32)
    off_ref[...] = poff[0].astype(jnp.int32)
    cnt_ref[...] = counts


def _router(x_flat, wg):
    return pl.pallas_call(
        _router_body,
        out_shape=[
            jax.ShapeDtypeStruct((_S,), jnp.int32),
            jax.ShapeDtypeStruct((_E,), jnp.int32),
            jax.ShapeDtypeStruct((_E,), jnp.int32),
        ],
    )(x_flat, wg)


def _ffn_body(off_ref, cnt_ref, x_ref, w1_ref, w2_ref, w3_ref, out_ref):
    e = pl.program_id(0)
    j = pl.program_id(1)
    start = off_ref[e]
    cnt = cnt_ref[e]
    trip = (cnt + _BM - 1) // _BM
    w1 = w1_ref[0]
    w2 = w2_ref[0]
    w3 = w3_ref[0]

    def chunk(k, carry):
        row = pl.multiple_of(start + k * _BM, _ALIGN)
        xs = x_ref[pl.ds(row, _BM), :]
        h1 = lax.dot_general(xs, w1, (((1,), (1,)), ((), ())),
                             preferred_element_type=jnp.float32)
        h2 = lax.dot_general(xs, w2, (((1,), (1,)), ((), ())),
                             preferred_element_type=jnp.float32)
        h = h1 * jax.nn.sigmoid(h1) * h2
        o = lax.dot_general(h, w3, (((1,), (1,)), ((), ())),
                            preferred_element_type=jnp.float32)

        @pl.when(j == 0)
        def _():
            out_ref[pl.ds(row, _BM), :] = o

        @pl.when(j != 0)
        def _():
            out_ref[pl.ds(row, _BM), :] = out_ref[pl.ds(row, _BM), :] + o

        return carry

    lax.fori_loop(0, trip, chunk, 0)


def _grouped_ffn(offsets, counts, x_sorted, w1, w2, w3):
    return pl.pallas_call(
        _ffn_body,
        grid=(_E, _HSPLIT),
        in_specs=[
            pl.BlockSpec(memory_space=pltpu.SMEM),
            pl.BlockSpec(memory_space=pltpu.SMEM),
            pl.BlockSpec((_P, _D), lambda e, j: (0, 0)),
            pl.BlockSpec((1, _H // _HSPLIT, _D), lambda e, j: (e, j, 0)),
            pl.BlockSpec((1, _H // _HSPLIT, _D), lambda e, j: (e, j, 0)),
            pl.BlockSpec((1, _D, _H // _HSPLIT), lambda e, j: (e, 0, j)),
        ],
        out_specs=pl.BlockSpec((_P, _D), lambda e, j: (0, 0)),
        out_shape=jax.ShapeDtypeStruct((_P, _D), jnp.float32),
    )(offsets, counts, x_sorted, w1, w2, w3)


def _sc_mesh():
    return plsc.VectorSubcoreMesh(core_axis_name="c", subcore_axis_name="s")


def _scatter_rows(x_flat, dest):
    n_per_w = _S // _NW

    @functools.partial(
        pl.kernel,
        out_type=jax.ShapeDtypeStruct((_P, _D), jnp.float32),
        mesh=_sc_mesh(),
        scratch_types=[
            pltpu.VMEM((n_per_w,), jnp.int32),
            pltpu.VMEM((n_per_w, _D), jnp.float32),
            pltpu.SemaphoreType.DMA,
        ],
    )
    def body(x_hbm, dest_hbm, out_hbm, idx_v, rows_v, sem):
        wid = lax.axis_index("s") * 2 + lax.axis_index("c")
        base = wid * n_per_w
        pltpu.sync_copy(dest_hbm.at[pl.ds(base, n_per_w)], idx_v)
        pltpu.sync_copy(x_hbm.at[pl.ds(base, n_per_w)], rows_v)
        pltpu.async_copy(rows_v, out_hbm.at[idx_v], sem).wait()

    return body(x_flat, dest)


def _gather_rows(table, dest):
    n_per_w = _S // _NW

    @functools.partial(
        pl.kernel,
        out_type=jax.ShapeDtypeStruct((_S, _D), jnp.float32),
        mesh=_sc_mesh(),
        scratch_types=[
            pltpu.VMEM((n_per_w,), jnp.int32),
            pltpu.VMEM((n_per_w, _D), jnp.float32),
            pltpu.SemaphoreType.DMA,
        ],
    )
    def body(table_hbm, dest_hbm, out_hbm, idx_v, rows_v, sem):
        wid = lax.axis_index("s") * 2 + lax.axis_index("c")
        base = wid * n_per_w
        pltpu.sync_copy(dest_hbm.at[pl.ds(base, n_per_w)], idx_v)
        pltpu.async_copy(table_hbm.at[idx_v], rows_v, sem).wait()
        pltpu.sync_copy(rows_v, out_hbm.at[pl.ds(base, n_per_w)])

    return body(table, dest)


def kernel(x, Wg, W1, W2, W3):
    b, s, d = x.shape
    x_flat = x.reshape(b * s, d)

    dest, padded_off, counts = _router(x_flat, Wg)

    x_sorted = _scatter_rows(x_flat, dest)
    out_sorted = _grouped_ffn(padded_off, counts, x_sorted, W1, W2, W3)
    out_flat = _gather_rows(out_sorted, dest)
    return out_flat.reshape(b, s, d)


# speedup vs baseline: 1.1182x; 1.1182x over previous
import functools

import jax
import jax.numpy as jnp
from jax import lax
from jax.experimental import pallas as pl
from jax.experimental.pallas import tpu as pltpu
from jax.experimental.pallas import tpu_sc as plsc

_E = 64
_D = 768
_H = 1024
_S = 2048

_ALIGN = 8
_BM = 32
_NW = 32
_P = 2816


def _router_body(x_ref, wg_ref, dest_ref, off_ref, cnt_ref):
    scores = lax.dot_general(x_ref[...], wg_ref[...],
                             (((1,), (1,)), ((), ())),
                             preferred_element_type=jnp.float32)
    idx = jnp.argmax(scores, axis=1).astype(jnp.int32)
    eids = lax.broadcasted_iota(jnp.int32, (1, _E), 1)
    onehot = (idx[:, None] == eids).astype(jnp.float32)
    r = lax.broadcasted_iota(jnp.int32, (_S, _S), 0)
    c = lax.broadcasted_iota(jnp.int32, (_S, _S), 1)
    tril = (c <= r).astype(jnp.float32)
    csum = lax.dot_general(tril, onehot, (((1,), (0,)), ((), ())),
                           preferred_element_type=jnp.float32)
    within = jnp.sum(csum * onehot, axis=1) - 1.0
    counts = csum[_S - 1, :].astype(jnp.int32)
    padded = (counts + (_ALIGN - 1)) // _ALIGN * _ALIGN
    re = lax.broadcasted_iota(jnp.int32, (_E, _E), 0)
    ce = lax.broadcasted_iota(jnp.int32, (_E, _E), 1)
    sut = (re < ce).astype(jnp.float32)
    poff = lax.dot_general(padded.astype(jnp.float32)[None, :], sut,
                           (((1,), (0,)), ((), ())),
                           preferred_element_type=jnp.float32)
    off_tok = jnp.sum(onehot * poff, axis=1)
    dest_ref[...] = (off_tok + within).astype(jnp.int32)
    off_ref[...] = poff[0].astype(jnp.int32)
    cnt_ref[...] = counts


def _router(x_flat, wg):
    return pl.pallas_call(
        _router_body,
        out_shape=[
            jax.ShapeDtypeStruct((_S,), jnp.int32),
            jax.ShapeDtypeStruct((_E,), jnp.int32),
            jax.ShapeDtypeStruct((_E,), jnp.int32),
        ],
    )(x_flat, wg)


def _ffn_body(off_ref, cnt_ref, x_ref, w1_ref, w2_ref, w3_ref, out_ref):
    e = pl.program_id(0)
    start = off_ref[e]
    cnt = cnt_ref[e]
    trip = (cnt + _BM - 1) // _BM
    w1 = w1_ref[0]
    w2 = w2_ref[0]
    w3 = w3_ref[0]

    def chunk(k, carry):
        row = pl.multiple_of(start + k * _BM, _ALIGN)
        xs = x_ref[pl.ds(row, _BM), :]
        h1 = lax.dot_general(xs, w1, (((1,), (1,)), ((), ())),
                             preferred_element_type=jnp.float32)
        h2 = lax.dot_general(xs, w2, (((1,), (1,)), ((), ())),
                             preferred_element_type=jnp.float32)
        h = h1 * jax.nn.sigmoid(h1) * h2
        o = lax.dot_general(h, w3, (((1,), (1,)), ((), ())),
                            preferred_element_type=jnp.float32)
        out_ref[pl.ds(row, _BM), :] = o
        return carry

    lax.fori_loop(0, trip, chunk, 0)


def _grouped_ffn(offsets, counts, x_sorted, w1, w2, w3):
    return pl.pallas_call(
        _ffn_body,
        grid=(_E,),
        in_specs=[
            pl.BlockSpec(memory_space=pltpu.SMEM),
            pl.BlockSpec(memory_space=pltpu.SMEM),
            pl.BlockSpec((_P, _D), lambda e: (0, 0)),
            pl.BlockSpec((1, _H, _D), lambda e: (e, 0, 0)),
            pl.BlockSpec((1, _H, _D), lambda e: (e, 0, 0)),
            pl.BlockSpec((1, _D, _H), lambda e: (e, 0, 0)),
        ],
        out_specs=pl.BlockSpec((_P, _D), lambda e: (0, 0)),
        out_shape=jax.ShapeDtypeStruct((_P, _D), jnp.float32),
    )(offsets, counts, x_sorted, w1, w2, w3)


def _sc_mesh():
    return plsc.VectorSubcoreMesh(core_axis_name="c", subcore_axis_name="s")


def _scatter_rows(x_flat, dest):
    n_per_w = _S // _NW

    @functools.partial(
        pl.kernel,
        out_type=jax.ShapeDtypeStruct((_P, _D), jnp.float32),
        mesh=_sc_mesh(),
        scratch_types=[
            pltpu.VMEM((n_per_w,), jnp.int32),
            pltpu.VMEM((n_per_w, _D), jnp.float32),
            pltpu.SemaphoreType.DMA,
        ],
    )
    def body(x_hbm, dest_hbm, out_hbm, idx_v, rows_v, sem):
        wid = lax.axis_index("s") * 2 + lax.axis_index("c")
        base = wid * n_per_w
        pltpu.sync_copy(dest_hbm.at[pl.ds(base, n_per_w)], idx_v)
        pltpu.sync_copy(x_hbm.at[pl.ds(base, n_per_w)], rows_v)
        pltpu.async_copy(rows_v, out_hbm.at[idx_v], sem).wait()

    return body(x_flat, dest)


def _gather_rows(table, dest):
    n_per_w = _S // _NW

    @functools.partial(
        pl.kernel,
        out_type=jax.ShapeDtypeStruct((_S, _D), jnp.float32),
        mesh=_sc_mesh(),
        scratch_types=[
            pltpu.VMEM((n_per_w,), jnp.int32),
            pltpu.VMEM((n_per_w, _D), jnp.float32),
            pltpu.SemaphoreType.DMA,
        ],
    )
    def body(table_hbm, dest_hbm, out_hbm, idx_v, rows_v, sem):
        wid = lax.axis_index("s") * 2 + lax.axis_index("c")
        base = wid * n_per_w
        pltpu.sync_copy(dest_hbm.at[pl.ds(base, n_per_w)], idx_v)
        pltpu.async_copy(table_hbm.at[idx_v], rows_v, sem).wait()
        pltpu.sync_copy(rows_v, out_hbm.at[pl.ds(base, n_per_w)])

    return body(table, dest)


def kernel(x, Wg, W1, W2, W3):
    b, s, d = x.shape
    x_flat = x.reshape(b * s, d)

    dest, padded_off, counts = _router(x_flat, Wg)

    x_sorted = _scatter_rows(x_flat, dest)
    out_sorted = _grouped_ffn(padded_off, counts, x_sorted, W1, W2, W3)
    out_flat = _gather_rows(out_sorted, dest)
    return out_flat.reshape(b, s, d)


# speedup vs baseline: 1.2741x; 1.1394x over previous
import functools

import jax
import jax.numpy as jnp
from jax import lax
from jax.experimental import pallas as pl
from jax.experimental.pallas import tpu as pltpu
from jax.experimental.pallas import tpu_sc as plsc

_E = 64
_D = 768
_H = 1024
_S = 2048

_ALIGN = 8
_BM = 64
_NW = 32
_P = 2816


def _router_body(x_ref, wg_ref, dest_ref, off_ref, cnt_ref):
    scores = lax.dot_general(x_ref[...], wg_ref[...],
                             (((1,), (1,)), ((), ())),
                             preferred_element_type=jnp.float32)
    idx = jnp.argmax(scores, axis=1).astype(jnp.int32)
    eids = lax.broadcasted_iota(jnp.int32, (1, _E), 1)
    onehot = (idx[:, None] == eids).astype(jnp.float32)
    r = lax.broadcasted_iota(jnp.int32, (_S, _S), 0)
    c = lax.broadcasted_iota(jnp.int32, (_S, _S), 1)
    tril = (c <= r).astype(jnp.float32)
    csum = lax.dot_general(tril, onehot, (((1,), (0,)), ((), ())),
                           preferred_element_type=jnp.float32)
    within = jnp.sum(csum * onehot, axis=1) - 1.0
    counts = csum[_S - 1, :].astype(jnp.int32)
    padded = (counts + (_ALIGN - 1)) // _ALIGN * _ALIGN
    re = lax.broadcasted_iota(jnp.int32, (_E, _E), 0)
    ce = lax.broadcasted_iota(jnp.int32, (_E, _E), 1)
    sut = (re < ce).astype(jnp.float32)
    poff = lax.dot_general(padded.astype(jnp.float32)[None, :], sut,
                           (((1,), (0,)), ((), ())),
                           preferred_element_type=jnp.float32)
    off_tok = jnp.sum(onehot * poff, axis=1)
    dest_ref[...] = (off_tok + within).astype(jnp.int32)
    off_ref[...] = poff[0].astype(jnp.int32)
    cnt_ref[...] = counts


def _router(x_flat, wg):
    return pl.pallas_call(
        _router_body,
        out_shape=[
            jax.ShapeDtypeStruct((_S,), jnp.int32),
            jax.ShapeDtypeStruct((_E,), jnp.int32),
            jax.ShapeDtypeStruct((_E,), jnp.int32),
        ],
    )(x_flat, wg)


def _ffn_body(off_ref, cnt_ref, x_ref, w1_ref, w2_ref, w3_ref, out_ref):
    e = pl.program_id(0)
    start = off_ref[e]
    cnt = cnt_ref[e]
    trip = (cnt + _BM - 1) // _BM
    w1 = w1_ref[0].astype(jnp.bfloat16)
    w2 = w2_ref[0].astype(jnp.bfloat16)
    w3 = w3_ref[0].astype(jnp.bfloat16)

    def chunk(k, carry):
        row = pl.multiple_of(start + k * _BM, _ALIGN)
        xs = x_ref[pl.ds(row, _BM), :].astype(jnp.bfloat16)
        h1 = lax.dot_general(xs, w1, (((1,), (1,)), ((), ())),
                             preferred_element_type=jnp.float32)
        h2 = lax.dot_general(xs, w2, (((1,), (1,)), ((), ())),
                             preferred_element_type=jnp.float32)
        h = (h1 * jax.nn.sigmoid(h1) * h2).astype(jnp.bfloat16)
        o = lax.dot_general(h, w3, (((1,), (1,)), ((), ())),
                            preferred_element_type=jnp.float32)
        out_ref[pl.ds(row, _BM), :] = o
        return carry

    lax.fori_loop(0, trip, chunk, 0)


def _grouped_ffn(offsets, counts, x_sorted, w1, w2, w3):
    return pl.pallas_call(
        _ffn_body,
        grid=(_E,),
        in_specs=[
            pl.BlockSpec(memory_space=pltpu.SMEM),
            pl.BlockSpec(memory_space=pltpu.SMEM),
            pl.BlockSpec((_P, _D), lambda e: (0, 0)),
            pl.BlockSpec((1, _H, _D), lambda e: (e, 0, 0)),
            pl.BlockSpec((1, _H, _D), lambda e: (e, 0, 0)),
            pl.BlockSpec((1, _D, _H), lambda e: (e, 0, 0)),
        ],
        out_specs=pl.BlockSpec((_P, _D), lambda e: (0, 0)),
        out_shape=jax.ShapeDtypeStruct((_P, _D), jnp.float32),
    )(offsets, counts, x_sorted, w1, w2, w3)


def _sc_mesh():
    return plsc.VectorSubcoreMesh(core_axis_name="c", subcore_axis_name="s")


def _scatter_rows(x_flat, dest):
    n_per_w = _S // _NW

    @functools.partial(
        pl.kernel,
        out_type=jax.ShapeDtypeStruct((_P, _D), jnp.float32),
        mesh=_sc_mesh(),
        scratch_types=[
            pltpu.VMEM((n_per_w,), jnp.int32),
            pltpu.VMEM((n_per_w, _D), jnp.float32),
            pltpu.SemaphoreType.DMA,
        ],
    )
    def body(x_hbm, dest_hbm, out_hbm, idx_v, rows_v, sem):
        wid = lax.axis_index("s") * 2 + lax.axis_index("c")
        base = wid * n_per_w
        pltpu.sync_copy(dest_hbm.at[pl.ds(base, n_per_w)], idx_v)
        pltpu.sync_copy(x_hbm.at[pl.ds(base, n_per_w)], rows_v)
        pltpu.async_copy(rows_v, out_hbm.at[idx_v], sem).wait()

    return body(x_flat, dest)


def _gather_rows(table, dest):
    n_per_w = _S // _NW

    @functools.partial(
        pl.kernel,
        out_type=jax.ShapeDtypeStruct((_S, _D), jnp.float32),
        mesh=_sc_mesh(),
        scratch_types=[
            pltpu.VMEM((n_per_w,), jnp.int32),
            pltpu.VMEM((n_per_w, _D), jnp.float32),
            pltpu.SemaphoreType.DMA,
        ],
    )
    def body(table_hbm, dest_hbm, out_hbm, idx_v, rows_v, sem):
        wid = lax.axis_index("s") * 2 + lax.axis_index("c")
        base = wid * n_per_w
        pltpu.sync_copy(dest_hbm.at[pl.ds(base, n_per_w)], idx_v)
        pltpu.async_copy(table_hbm.at[idx_v], rows_v, sem).wait()
        pltpu.sync_copy(rows_v, out_hbm.at[pl.ds(base, n_per_w)])

    return body(table, dest)


def kernel(x, Wg, W1, W2, W3):
    b, s, d = x.shape
    x_flat = x.reshape(b * s, d)

    dest, padded_off, counts = _router(x_flat, Wg)

    x_sorted = _scatter_rows(x_flat, dest)
    out_sorted = _grouped_ffn(padded_off, counts, x_sorted, W1, W2, W3)
    out_flat = _gather_rows(out_sorted, dest)
    return out_flat.reshape(b, s, d)
